# baseline (speedup 1.0000x reference)
import functools

import jax
import jax.numpy as jnp
from jax.experimental import pallas as pl
from jax.experimental.pallas import tpu as pltpu

_N = 262144
_K = 512
_D = 32
_BN = 16384


def _body(e_ref, cm2_ref, c2_ref, ids_ref, loss_ref):
    i = pl.program_id(0)
    ev = e_ref[...]
    cm2 = cm2_ref[...]
    c2 = c2_ref[...]
    iota8 = jax.lax.broadcasted_iota(jnp.int32, (8, _BN // 8), 0)
    part = jnp.sum(ev * ev)
    for q in range(8):
        s = jax.lax.dot_general(
            cm2, ev[32 * q:32 * q + 32, :], (((1,), (0,)), ((), ())),
            preferred_element_type=jnp.float32)
        s = s + c2
        runv = s[0:8, :]
        runi = iota8
        for r in range(1, _K // 8):
            v = s[8 * r:8 * r + 8, :]
            lt = v < runv
            runv = jnp.where(lt, v, runv)
            runi = jnp.where(lt, iota8 + 8 * r, runi)
        m8 = jnp.min(runv, axis=0, keepdims=True)
        sel = jnp.where(runv == m8, runi, _K)
        ids_ref[q, :] = jnp.min(sel, axis=0)
        part = part + jnp.sum(m8)

    @pl.when(i == 0)
    def _():
        loss_ref[0, 0] = 0.0

    loss_ref[0, 0] += part


@jax.jit
def _cluster(embs, centers):
    cm2 = -2.0 * centers
    c2 = jnp.sum(centers * centers, axis=1, keepdims=True)
    ev = embs.reshape(_N // 8, 256).T
    grid = _N // _BN
    ids, loss = pl.pallas_call(
        _body,
        grid=(grid,),
        in_specs=[
            pl.BlockSpec((256, _BN // 8), lambda i: (0, i)),
            pl.BlockSpec((_K, _D), lambda i: (0, 0)),
            pl.BlockSpec((_K, 1), lambda i: (0, 0)),
        ],
        out_specs=[
            pl.BlockSpec((8, _BN // 8), lambda i: (0, i)),
            pl.BlockSpec((1, 1), lambda i: (0, 0), memory_space=pltpu.SMEM),
        ],
        out_shape=[
            jax.ShapeDtypeStruct((8, _N // 8), jnp.int32),
            jax.ShapeDtypeStruct((1, 1), jnp.float32),
        ],
    )(ev, cm2, c2)
    return ids.T.reshape(_N), loss[0, 0]


def kernel(embs, centers):
    ids, loss = _cluster(embs, centers)
    return (centers, ids, loss)


# speedup vs baseline: 2.8711x; 2.8711x over previous
import jax
import jax.numpy as jnp
from jax.experimental import pallas as pl
from jax.experimental.pallas import tpu as pltpu

_N = 262144
_K = 512
_D = 32
_BN = 16384


def _body(e_ref, cm2_ref, c2_ref, ids_ref, loss_ref):
    i = pl.program_id(0)
    e = e_ref[...]
    cm2 = cm2_ref[...]
    s = jax.lax.dot_general(
        cm2, e, (((1,), (0,)), ((), ())),
        preferred_element_type=jnp.float32)
    s = s + c2_ref[...]
    iota8 = jax.lax.broadcasted_iota(jnp.int32, (8, _BN), 0)
    runv = s[0:8, :]
    runi = iota8
    for r in range(1, _K // 8):
        v = s[8 * r:8 * r + 8, :]
        lt = v < runv
        runv = jnp.where(lt, v, runv)
        runi = jnp.where(lt, iota8 + 8 * r, runi)
    m8 = jnp.min(runv, axis=0, keepdims=True)
    sel = jnp.where(runv == m8, runi, _K)
    ids_ref[...] = jnp.min(sel, axis=0)

    part = jnp.sum(e * e) + jnp.sum(m8)

    @pl.when(i == 0)
    def _():
        loss_ref[0, 0] = 0.0

    loss_ref[0, 0] += part


@jax.jit
def _cluster(embs, centers):
    cm2 = -2.0 * centers
    c2 = jnp.sum(centers * centers, axis=1, keepdims=True)
    eT = embs.T
    grid = _N // _BN
    ids, loss = pl.pallas_call(
        _body,
        grid=(grid,),
        in_specs=[
            pl.BlockSpec((_D, _BN), lambda i: (0, i)),
            pl.BlockSpec((_K, _D), lambda i: (0, 0)),
            pl.BlockSpec((_K, 1), lambda i: (0, 0)),
        ],
        out_specs=[
            pl.BlockSpec((_BN,), lambda i: (i,)),
            pl.BlockSpec((1, 1), lambda i: (0, 0), memory_space=pltpu.SMEM),
        ],
        out_shape=[
            jax.ShapeDtypeStruct((_N,), jnp.int32),
            jax.ShapeDtypeStruct((1, 1), jnp.float32),
        ],
    )(eT, cm2, c2)
    return ids, loss[0, 0]


def kernel(embs, centers):
    ids, loss = _cluster(embs, centers)
    return (centers, ids, loss)
